# Initial kernel scaffold; baseline (speedup 1.0000x reference)
import jax
import jax.numpy as jnp
from jax.experimental import pallas as pl


def kernel(x, pe, text_duration, train):
    raise NotImplementedError("write your pallas kernel here")


# speedup vs baseline: 12.6434x; 12.6434x over previous
import jax
import jax.numpy as jnp
from jax.experimental import pallas as pl


def _pe_add_block(dur_ref, pe_ref, x_ref, o_ref, *, sblk, batches, words):
    b = pl.program_id(0)
    sidx = pl.program_id(1)

    dur = dur_ref[...]
    bsel = jax.lax.broadcasted_iota(jnp.int32, (batches, words), 0) == b
    dur_b = jnp.sum(jnp.where(bsel, dur, 0), axis=0, keepdims=True)
    dur_bf = dur_b.astype(jnp.float32)

    tri = (
        jax.lax.broadcasted_iota(jnp.int32, (words, words), 0)
        <= jax.lax.broadcasted_iota(jnp.int32, (words, words), 1)
    ).astype(jnp.float32)
    csum_in = jnp.dot(dur_bf, tri, preferred_element_type=jnp.float32)
    csum_ex = csum_in - dur_bf

    pos = (
        jax.lax.broadcasted_iota(jnp.int32, (sblk, words), 0) + sidx * sblk
    ).astype(jnp.float32)
    onehot = ((pos >= csum_ex) & (pos < csum_in)).astype(jnp.float32)

    add = jnp.dot(onehot, pe_ref[...], preferred_element_type=jnp.float32)
    o_ref[...] = x_ref[...] + add


def kernel(x, pe, text_duration, train):
    del train
    S, B, C = x.shape
    _, W = text_duration.shape
    pe_trunc = pe[:W, :]
    sblk = 512
    grid = (B, S // sblk)

    x2 = x.reshape(S, B * C)
    out = pl.pallas_call(
        lambda dur_ref, pe_ref, x_ref, o_ref: _pe_add_block(
            dur_ref, pe_ref, x_ref, o_ref, sblk=sblk, batches=B, words=W
        ),
        grid=grid,
        in_specs=[
            pl.BlockSpec((B, W), lambda b, s: (0, 0)),
            pl.BlockSpec((W, C), lambda b, s: (0, 0)),
            pl.BlockSpec((sblk, C), lambda b, s: (s, b)),
        ],
        out_specs=pl.BlockSpec((sblk, C), lambda b, s: (s, b)),
        out_shape=jax.ShapeDtypeStruct((S, B * C), x.dtype),
    )(text_duration.astype(jnp.int32), pe_trunc, x2)
    return out.reshape(S, B, C)


# speedup vs baseline: 45.1166x; 3.5684x over previous
import functools

import jax
import jax.numpy as jnp
from jax.experimental import pallas as pl


def _pe_add_block(dur_ref, pe_ref, x_ref, o_ref, *, sblk, batches, words):
    sidx = pl.program_id(0)

    dur = dur_ref[...].astype(jnp.float32)
    tri = (
        jax.lax.broadcasted_iota(jnp.int32, (words, words), 0)
        <= jax.lax.broadcasted_iota(jnp.int32, (words, words), 1)
    ).astype(jnp.float32)
    csum_in = jnp.dot(dur, tri, preferred_element_type=jnp.float32)
    csum_ex = csum_in - dur

    pos = (
        jax.lax.broadcasted_iota(jnp.int32, (sblk, words), 0) + sidx * sblk
    ).astype(jnp.float32)

    pe_bf = pe_ref[...].astype(jnp.bfloat16)
    for b in range(batches):
        onehot = (
            (pos >= csum_ex[b : b + 1, :]) & (pos < csum_in[b : b + 1, :])
        ).astype(jnp.bfloat16)
        add = jnp.dot(onehot, pe_bf, preferred_element_type=jnp.float32)
        o_ref[:, b, :] = x_ref[:, b, :] + add


def kernel(x, pe, text_duration, train):
    del train
    S, B, C = x.shape
    _, W = text_duration.shape
    pe_trunc = pe[:W, :]
    sblk = 256
    grid = (S // sblk,)

    return pl.pallas_call(
        functools.partial(_pe_add_block, sblk=sblk, batches=B, words=W),
        grid=grid,
        in_specs=[
            pl.BlockSpec((B, W), lambda s: (0, 0)),
            pl.BlockSpec((W, C), lambda s: (0, 0)),
            pl.BlockSpec((sblk, B, C), lambda s: (s, 0, 0)),
        ],
        out_specs=pl.BlockSpec((sblk, B, C), lambda s: (s, 0, 0)),
        out_shape=jax.ShapeDtypeStruct((S, B, C), x.dtype),
    )(text_duration.astype(jnp.int32), pe_trunc, x)


# speedup vs baseline: 51.8481x; 1.1492x over previous
import functools

import jax
import jax.numpy as jnp
from jax.experimental import pallas as pl


def _pe_add_block(dur_ref, pe_ref, x_ref, o_ref, *, sblk, batches, words):
    sidx = pl.program_id(0)
    rows = sblk * batches

    dur = dur_ref[...].astype(jnp.float32)
    tri = (
        jax.lax.broadcasted_iota(jnp.int32, (words, words), 0)
        <= jax.lax.broadcasted_iota(jnp.int32, (words, words), 1)
    ).astype(jnp.float32)
    csum_in = jnp.dot(dur, tri, preferred_element_type=jnp.float32)
    csum_ex = csum_in - dur

    ci_t = jnp.broadcast_to(csum_in[None], (sblk, batches, words)).reshape(
        rows, words
    )
    ce_t = jnp.broadcast_to(csum_ex[None], (sblk, batches, words)).reshape(
        rows, words
    )

    pos = (
        (jax.lax.broadcasted_iota(jnp.int32, (rows, words), 0) // batches)
        + sidx * sblk
    ).astype(jnp.float32)

    onehot = ((pos >= ce_t) & (pos < ci_t)).astype(jnp.bfloat16)
    pe_bf = pe_ref[...].astype(jnp.bfloat16)
    add = jnp.dot(onehot, pe_bf, preferred_element_type=jnp.float32)

    chans = pe_ref.shape[1]
    xb = x_ref[...].reshape(rows, chans)
    o_ref[...] = (xb + add).reshape(sblk, batches, chans)


def kernel(x, pe, text_duration, train):
    del train
    S, B, C = x.shape
    _, W = text_duration.shape
    pe_trunc = pe[:W, :]
    sblk = 128
    grid = (S // sblk,)

    return pl.pallas_call(
        functools.partial(_pe_add_block, sblk=sblk, batches=B, words=W),
        grid=grid,
        in_specs=[
            pl.BlockSpec((B, W), lambda s: (0, 0)),
            pl.BlockSpec((W, C), lambda s: (0, 0)),
            pl.BlockSpec((sblk, B, C), lambda s: (s, 0, 0)),
        ],
        out_specs=pl.BlockSpec((sblk, B, C), lambda s: (s, 0, 0)),
        out_shape=jax.ShapeDtypeStruct((S, B, C), x.dtype),
    )(text_duration.astype(jnp.int32), pe_trunc, x)


# speedup vs baseline: 53.6143x; 1.0341x over previous
import functools

import jax
import jax.numpy as jnp
from jax.experimental import pallas as pl


def _pe_add_block(dur_ref, pe_ref, x_ref, o_ref, *, sblk, batches, words):
    sidx = pl.program_id(0)
    rows = sblk * batches

    dur = dur_ref[...].astype(jnp.float32)
    tri = (
        jax.lax.broadcasted_iota(jnp.int32, (words, words), 0)
        <= jax.lax.broadcasted_iota(jnp.int32, (words, words), 1)
    ).astype(jnp.float32)
    csum_in = jnp.dot(dur, tri, preferred_element_type=jnp.float32)
    csum_ex = csum_in - dur

    ci_t = jnp.broadcast_to(csum_in[None], (sblk, batches, words)).reshape(
        rows, words
    )
    ce_t = jnp.broadcast_to(csum_ex[None], (sblk, batches, words)).reshape(
        rows, words
    )

    pos = (
        (jax.lax.broadcasted_iota(jnp.int32, (rows, words), 0) // batches)
        + sidx * sblk
    ).astype(jnp.float32)

    onehot = ((pos >= ce_t) & (pos < ci_t)).astype(jnp.bfloat16)
    pe_bf = pe_ref[...].astype(jnp.bfloat16)
    add = jnp.dot(onehot, pe_bf, preferred_element_type=jnp.float32)

    chans = pe_ref.shape[1]
    xb = x_ref[...].reshape(rows, chans)
    o_ref[...] = (xb + add).reshape(sblk, batches, chans)


def kernel(x, pe, text_duration, train):
    del train
    S, B, C = x.shape
    _, W = text_duration.shape
    pe_trunc = pe[:W, :]
    sblk = 256
    grid = (S // sblk,)

    return pl.pallas_call(
        functools.partial(_pe_add_block, sblk=sblk, batches=B, words=W),
        grid=grid,
        in_specs=[
            pl.BlockSpec((B, W), lambda s: (0, 0)),
            pl.BlockSpec((W, C), lambda s: (0, 0)),
            pl.BlockSpec((sblk, B, C), lambda s: (s, 0, 0)),
        ],
        out_specs=pl.BlockSpec((sblk, B, C), lambda s: (s, 0, 0)),
        out_shape=jax.ShapeDtypeStruct((S, B, C), x.dtype),
    )(text_duration.astype(jnp.int32), pe_trunc, x)
